# baseline (speedup 1.0000x reference)
import functools

import jax
import jax.numpy as jnp
from jax import lax
from jax.experimental import pallas as pl
from jax.experimental.pallas import tpu as pltpu
from jax.experimental.pallas import tpu_sc as plsc

T = 2048
D = 768
F = 3072
E = 8
K = 2
BLOCK_M = 128
MSHIFT = 7
NT = (T * K) // BLOCK_M + E
M_PAD = NT * BLOCK_M
NSLOT = T * K
NR = NSLOT // 128

_MESH = plsc.VectorSubcoreMesh(core_axis_name="c", subcore_axis_name="s")

_HI = lax.Precision.HIGHEST


def _meta_body(ids_ref, pos_ref, eot_ref, rows_ref):
    f32, i32 = jnp.float32, jnp.int32
    ids = ids_ref[...]
    tri_incl = (lax.broadcasted_iota(i32, (128, 128), 0)
                <= lax.broadcasted_iota(i32, (128, 128), 1)).astype(f32)
    tri_strict = (lax.broadcasted_iota(i32, (NR, NR), 0)
                  < lax.broadcasted_iota(i32, (NR, NR), 1)).astype(f32)
    tri8 = (lax.broadcasted_iota(i32, (E, E), 0)
            <= lax.broadcasted_iota(i32, (E, E), 1)).astype(f32)

    masks, prefs, rowtots = [], [], []
    for e in range(E):
        m = (ids == e).astype(f32)
        p = lax.dot_general(m, tri_incl, (((1,), (0,)), ((), ())),
                            precision=_HI, preferred_element_type=f32)
        masks.append(m)
        prefs.append(p)
        rowtots.append(p[:, 127:128])
    rowtot = jnp.concatenate(rowtots, axis=1)
    excl = lax.dot_general(tri_strict, rowtot, (((0,), (0,)), ((), ())),
                           precision=_HI, preferred_element_type=f32)
    counts = excl[NR - 1:NR, :] + rowtot[NR - 1:NR, :]
    counts_i = counts.astype(i32)
    tiles_i = (counts_i + (BLOCK_M - 1)) >> MSHIFT
    tiles_f = tiles_i.astype(f32)
    cumt = lax.dot_general(tiles_f, tri8, (((1,), (0,)), ((), ())),
                           precision=_HI, preferred_element_type=f32)
    off_f = cumt - tiles_f

    pos = jnp.zeros((NR, 128), f32)
    for e in range(E):
        base_e = excl[:, e:e + 1] + off_f[0:1, e:e + 1] * BLOCK_M
        pos = pos + masks[e] * (prefs[e] - 1.0 + base_e)
    pos_ref[...] = pos.astype(i32)

    ti = lax.broadcasted_iota(i32, (1, 128), 1).astype(f32)
    eot = jnp.zeros((1, 128), f32)
    for e in range(E - 1):
        eot = eot + (ti >= cumt[0:1, e:e + 1]).astype(f32)
    eot_i = jnp.minimum(eot.astype(i32), E - 1)
    rows = jnp.zeros((1, 128), f32)
    for e in range(E):
        rows_e = counts[0:1, e:e + 1] - (ti - off_f[0:1, e:e + 1]) * BLOCK_M
        rows = rows + (eot_i == e).astype(f32) * rows_e
    eot_ref[...] = eot_i
    rows_ref[...] = jnp.clip(rows.astype(i32), 0, BLOCK_M)


_meta = pl.pallas_call(
    _meta_body,
    out_shape=(
        jax.ShapeDtypeStruct((NR, 128), jnp.int32),
        jax.ShapeDtypeStruct((1, 128), jnp.int32),
        jax.ShapeDtypeStruct((1, 128), jnp.int32),
    ),
)


def _disp_body(hs_hbm, pos_hbm, tok_hbm, w_hbm, xpad_hbm, ws_hbm,
               tokbuf, posbuf, posA, posB, wbuf, rows0, rows1, gsem, ssem):
    c = lax.axis_index("c")
    s = lax.axis_index("s")
    wid = s * 2 + c
    base = wid * 128
    pltpu.sync_copy(pos_hbm.at[pl.ds(base, 128)], posbuf)
    pltpu.sync_copy(pos_hbm.at[pl.ds(base, 64)], posA)
    pltpu.sync_copy(pos_hbm.at[pl.ds(base + 64, 64)], posB)
    pltpu.sync_copy(tok_hbm.at[pl.ds(base, 128)], tokbuf)
    pltpu.sync_copy(w_hbm.at[pl.ds(base, 128)], wbuf)
    g0 = pltpu.make_async_copy(hs_hbm.at[tokbuf.at[pl.ds(0, 64)]], rows0, gsem)
    g1 = pltpu.make_async_copy(hs_hbm.at[tokbuf.at[pl.ds(64, 64)]], rows1, gsem)
    g0.start()
    g1.start()
    g0.wait()
    s0 = pltpu.make_async_copy(rows0, xpad_hbm.at[posA], ssem)
    s0.start()
    g1.wait()
    s1 = pltpu.make_async_copy(rows1, xpad_hbm.at[posB], ssem)
    s1.start()
    pltpu.sync_copy(wbuf, ws_hbm.at[posbuf])
    s0.wait()
    s1.wait()


_dispatch = functools.partial(
    pl.kernel,
    out_type=(
        jax.ShapeDtypeStruct((M_PAD, D), jnp.float32),
        jax.ShapeDtypeStruct((M_PAD,), jnp.float32),
    ),
    mesh=_MESH,
    scratch_types=[
        pltpu.VMEM((128,), jnp.int32),
        pltpu.VMEM((128,), jnp.int32),
        pltpu.VMEM((64,), jnp.int32),
        pltpu.VMEM((64,), jnp.int32),
        pltpu.VMEM((128,), jnp.float32),
        pltpu.VMEM((64, D), jnp.float32),
        pltpu.VMEM((64, D), jnp.float32),
        pltpu.SemaphoreType.DMA,
        pltpu.SemaphoreType.DMA,
    ],
)(_disp_body)


def _gu_body(eot_ref, rows_ref, x_ref, wg_ref, wu_ref, g_ref):
    m = pl.program_id(0)

    @pl.when(rows_ref[m] > 0)
    def _():
        x = x_ref[...]
        hg = lax.dot_general(x, wg_ref[0], (((1,), (1,)), ((), ())),
                             preferred_element_type=jnp.float32)
        hu = lax.dot_general(x, wu_ref[0], (((1,), (1,)), ((), ())),
                             preferred_element_type=jnp.float32)
        g_ref[...] = hg * jax.nn.sigmoid(hg) * hu


_grouped_gu = pl.pallas_call(
    _gu_body,
    grid_spec=pltpu.PrefetchScalarGridSpec(
        num_scalar_prefetch=2,
        grid=(NT,),
        in_specs=[
            pl.BlockSpec((BLOCK_M, D), lambda m, eot, rows: (m, 0)),
            pl.BlockSpec((1, F, D), lambda m, eot, rows: (eot[m], 0, 0)),
            pl.BlockSpec((1, F, D), lambda m, eot, rows: (eot[m], 1, 0)),
        ],
        out_specs=pl.BlockSpec((BLOCK_M, F), lambda m, eot, rows: (m, 0)),
    ),
    out_shape=jax.ShapeDtypeStruct((M_PAD, F), jnp.float32),
)


def _down_body(eot_ref, rows_ref, g_ref, wd_ref, ws_ref, o_ref):
    m = pl.program_id(0)

    @pl.when(rows_ref[m] > 0)
    def _():
        o = lax.dot_general(g_ref[...], wd_ref[0], (((1,), (1,)), ((), ())),
                            preferred_element_type=jnp.float32)
        o_ref[...] = o * ws_ref[0, 0, :][:, None]


_grouped_down = pl.pallas_call(
    _down_body,
    grid_spec=pltpu.PrefetchScalarGridSpec(
        num_scalar_prefetch=2,
        grid=(NT,),
        in_specs=[
            pl.BlockSpec((BLOCK_M, F), lambda m, eot, rows: (m, 0)),
            pl.BlockSpec((1, D, F), lambda m, eot, rows: (eot[m], 0, 0)),
            pl.BlockSpec((1, 1, BLOCK_M), lambda m, eot, rows: (m, 0, 0)),
        ],
        out_specs=pl.BlockSpec((BLOCK_M, D), lambda m, eot, rows: (m, 0)),
    ),
    out_shape=jax.ShapeDtypeStruct((M_PAD, D), jnp.float32),
)


def _comb_body(y_hbm, pos_hbm, out_hbm, idx0, idx1, rows0, rows1, gsem, wsem):
    c = lax.axis_index("c")
    s = lax.axis_index("s")
    wid = s * 2 + c
    base = wid * 128
    pltpu.sync_copy(pos_hbm.at[pl.ds(base, 64)], idx0)
    pltpu.sync_copy(pos_hbm.at[pl.ds(base + 64, 64)], idx1)
    g0 = pltpu.make_async_copy(y_hbm.at[idx0], rows0, gsem)
    g1 = pltpu.make_async_copy(y_hbm.at[idx1], rows1, gsem)
    g0.start()
    g1.start()

    def compact(rows):
        def tok_body(i, _):
            for k in range(D // 16):
                rows[i, pl.ds(16 * k, 16)] = (
                    rows[2 * i, pl.ds(16 * k, 16)]
                    + rows[2 * i + 1, pl.ds(16 * k, 16)])
            return 0
        lax.fori_loop(0, 32, tok_body, 0)

    g0.wait()
    compact(rows0)
    w0 = pltpu.make_async_copy(rows0.at[pl.ds(0, 32)],
                               out_hbm.at[pl.ds(wid * 64, 32)], wsem)
    w0.start()
    g1.wait()
    compact(rows1)
    w1 = pltpu.make_async_copy(rows1.at[pl.ds(0, 32)],
                               out_hbm.at[pl.ds(wid * 64 + 32, 32)], wsem)
    w1.start()
    w0.wait()
    w1.wait()


_combine = functools.partial(
    pl.kernel,
    out_type=jax.ShapeDtypeStruct((T, D), jnp.float32),
    mesh=_MESH,
    scratch_types=[
        pltpu.VMEM((64,), jnp.int32),
        pltpu.VMEM((64,), jnp.int32),
        pltpu.VMEM((64, D), jnp.float32),
        pltpu.VMEM((64, D), jnp.float32),
        pltpu.SemaphoreType.DMA,
        pltpu.SemaphoreType.DMA,
    ],
)(_comb_body)


def kernel(hidden_states, topk_weights, topk_ids, gate_up_weights, down_weights):
    ids2d = topk_ids.reshape(NR, 128).astype(jnp.int32)
    w_flat = topk_weights.reshape(-1)
    tok_flat = jnp.arange(NSLOT, dtype=jnp.int32) // K

    pos2d, eot2d, rows2d = _meta(ids2d)
    pos = pos2d.reshape(NSLOT)
    eot = eot2d.reshape(128)
    rows = rows2d.reshape(128)

    x_pad, ws = _dispatch(hidden_states, pos, tok_flat, w_flat)
    g = _grouped_gu(eot, rows, x_pad, gate_up_weights, gate_up_weights)
    y = _grouped_down(eot, rows, g, down_weights, ws.reshape(NT, 1, BLOCK_M))
    out = _combine(y, pos)
    return out


# speedup vs baseline: 1.2866x; 1.2866x over previous
import functools

import jax
import jax.numpy as jnp
from jax import lax
from jax.experimental import pallas as pl
from jax.experimental.pallas import tpu as pltpu
from jax.experimental.pallas import tpu_sc as plsc

T = 2048
D = 768
F = 3072
E = 8
K = 2
BLOCK_M = 256
MSHIFT = 8
NT = (T * K) // BLOCK_M + E
M_PAD = NT * BLOCK_M
NSLOT = T * K
NR = NSLOT // 128

_MESH = plsc.VectorSubcoreMesh(core_axis_name="c", subcore_axis_name="s")

_HI = lax.Precision.HIGHEST


def _meta_body(ids_ref, pos_ref, eot_ref, rows_ref):
    f32, i32 = jnp.float32, jnp.int32
    ids = ids_ref[...]
    tri_incl = (lax.broadcasted_iota(i32, (128, 128), 0)
                <= lax.broadcasted_iota(i32, (128, 128), 1)).astype(f32)
    tri_strict = (lax.broadcasted_iota(i32, (NR, NR), 0)
                  < lax.broadcasted_iota(i32, (NR, NR), 1)).astype(f32)
    tri8 = (lax.broadcasted_iota(i32, (E, E), 0)
            <= lax.broadcasted_iota(i32, (E, E), 1)).astype(f32)

    masks, prefs, rowtots = [], [], []
    for e in range(E):
        m = (ids == e).astype(f32)
        p = lax.dot_general(m, tri_incl, (((1,), (0,)), ((), ())),
                            precision=_HI, preferred_element_type=f32)
        masks.append(m)
        prefs.append(p)
        rowtots.append(p[:, 127:128])
    rowtot = jnp.concatenate(rowtots, axis=1)
    excl = lax.dot_general(tri_strict, rowtot, (((0,), (0,)), ((), ())),
                           precision=_HI, preferred_element_type=f32)
    counts = excl[NR - 1:NR, :] + rowtot[NR - 1:NR, :]
    counts_i = counts.astype(i32)
    tiles_i = (counts_i + (BLOCK_M - 1)) >> MSHIFT
    tiles_f = tiles_i.astype(f32)
    cumt = lax.dot_general(tiles_f, tri8, (((1,), (0,)), ((), ())),
                           precision=_HI, preferred_element_type=f32)
    off_f = cumt - tiles_f

    pos = jnp.zeros((NR, 128), f32)
    for e in range(E):
        base_e = excl[:, e:e + 1] + off_f[0:1, e:e + 1] * BLOCK_M
        pos = pos + masks[e] * (prefs[e] - 1.0 + base_e)
    pos_ref[...] = pos.astype(i32)

    ti = lax.broadcasted_iota(i32, (1, 128), 1).astype(f32)
    eot = jnp.zeros((1, 128), f32)
    for e in range(E - 1):
        eot = eot + (ti >= cumt[0:1, e:e + 1]).astype(f32)
    eot_i = jnp.minimum(eot.astype(i32), E - 1)
    rows = jnp.zeros((1, 128), f32)
    for e in range(E):
        rows_e = counts[0:1, e:e + 1] - (ti - off_f[0:1, e:e + 1]) * BLOCK_M
        rows = rows + (eot_i == e).astype(f32) * rows_e
    eot_ref[...] = eot_i
    rows_ref[...] = jnp.clip(rows.astype(i32), 0, BLOCK_M)


_meta = pl.pallas_call(
    _meta_body,
    out_shape=(
        jax.ShapeDtypeStruct((NR, 128), jnp.int32),
        jax.ShapeDtypeStruct((1, 128), jnp.int32),
        jax.ShapeDtypeStruct((1, 128), jnp.int32),
    ),
)


def _disp_body(hs_hbm, pos_hbm, tok_hbm, w_hbm, xpad_hbm, ws_hbm,
               tokbuf, posbuf, posA, posB, wbuf, rows0, rows1, gsem, ssem):
    c = lax.axis_index("c")
    s = lax.axis_index("s")
    wid = s * 2 + c
    base = wid * 128
    pltpu.sync_copy(pos_hbm.at[pl.ds(base, 128)], posbuf)
    pltpu.sync_copy(pos_hbm.at[pl.ds(base, 64)], posA)
    pltpu.sync_copy(pos_hbm.at[pl.ds(base + 64, 64)], posB)
    pltpu.sync_copy(tok_hbm.at[pl.ds(base, 128)], tokbuf)
    pltpu.sync_copy(w_hbm.at[pl.ds(base, 128)], wbuf)
    g0 = pltpu.make_async_copy(hs_hbm.at[tokbuf.at[pl.ds(0, 64)]], rows0, gsem)
    g1 = pltpu.make_async_copy(hs_hbm.at[tokbuf.at[pl.ds(64, 64)]], rows1, gsem)
    g0.start()
    g1.start()
    g0.wait()
    s0 = pltpu.make_async_copy(rows0, xpad_hbm.at[posA], ssem)
    s0.start()
    g1.wait()
    s1 = pltpu.make_async_copy(rows1, xpad_hbm.at[posB], ssem)
    s1.start()
    pltpu.sync_copy(wbuf, ws_hbm.at[posbuf])
    s0.wait()
    s1.wait()


_dispatch = functools.partial(
    pl.kernel,
    out_type=(
        jax.ShapeDtypeStruct((M_PAD, D), jnp.float32),
        jax.ShapeDtypeStruct((M_PAD,), jnp.float32),
    ),
    mesh=_MESH,
    scratch_types=[
        pltpu.VMEM((128,), jnp.int32),
        pltpu.VMEM((128,), jnp.int32),
        pltpu.VMEM((64,), jnp.int32),
        pltpu.VMEM((64,), jnp.int32),
        pltpu.VMEM((128,), jnp.float32),
        pltpu.VMEM((64, D), jnp.float32),
        pltpu.VMEM((64, D), jnp.float32),
        pltpu.SemaphoreType.DMA,
        pltpu.SemaphoreType.DMA,
    ],
)(_disp_body)


def _gu_body(eot_ref, rows_ref, x_ref, wg_ref, wu_ref, g_ref):
    m = pl.program_id(0)

    @pl.when(rows_ref[m] > 0)
    def _():
        x = x_ref[...]
        hg = lax.dot_general(x, wg_ref[0], (((1,), (1,)), ((), ())),
                             preferred_element_type=jnp.float32)
        hu = lax.dot_general(x, wu_ref[0], (((1,), (1,)), ((), ())),
                             preferred_element_type=jnp.float32)
        g_ref[...] = hg * jax.nn.sigmoid(hg) * hu


_grouped_gu = pl.pallas_call(
    _gu_body,
    grid_spec=pltpu.PrefetchScalarGridSpec(
        num_scalar_prefetch=2,
        grid=(NT,),
        in_specs=[
            pl.BlockSpec((BLOCK_M, D), lambda m, eot, rows: (m, 0)),
            pl.BlockSpec((1, F, D), lambda m, eot, rows: (eot[m], 0, 0)),
            pl.BlockSpec((1, F, D), lambda m, eot, rows: (eot[m], 1, 0)),
        ],
        out_specs=pl.BlockSpec((BLOCK_M, F), lambda m, eot, rows: (m, 0)),
    ),
    out_shape=jax.ShapeDtypeStruct((M_PAD, F), jnp.float32),
)


def _down_body(eot_ref, rows_ref, g_ref, wd_ref, ws_ref, o_ref):
    m = pl.program_id(0)

    @pl.when(rows_ref[m] > 0)
    def _():
        o = lax.dot_general(g_ref[...], wd_ref[0], (((1,), (1,)), ((), ())),
                            preferred_element_type=jnp.float32)
        o_ref[...] = o * ws_ref[0, 0, :][:, None]


_grouped_down = pl.pallas_call(
    _down_body,
    grid_spec=pltpu.PrefetchScalarGridSpec(
        num_scalar_prefetch=2,
        grid=(NT,),
        in_specs=[
            pl.BlockSpec((BLOCK_M, F), lambda m, eot, rows: (m, 0)),
            pl.BlockSpec((1, D, F), lambda m, eot, rows: (eot[m], 0, 0)),
            pl.BlockSpec((1, 1, BLOCK_M), lambda m, eot, rows: (m, 0, 0)),
        ],
        out_specs=pl.BlockSpec((BLOCK_M, D), lambda m, eot, rows: (m, 0)),
    ),
    out_shape=jax.ShapeDtypeStruct((M_PAD, D), jnp.float32),
)


def _comb_body(y_hbm, pos_hbm, out_hbm, idx0, idx1, rows0, rows1, gsem, wsem):
    c = lax.axis_index("c")
    s = lax.axis_index("s")
    wid = s * 2 + c
    base = wid * 128
    pltpu.sync_copy(pos_hbm.at[pl.ds(base, 64)], idx0)
    pltpu.sync_copy(pos_hbm.at[pl.ds(base + 64, 64)], idx1)
    g0 = pltpu.make_async_copy(y_hbm.at[idx0], rows0, gsem)
    g1 = pltpu.make_async_copy(y_hbm.at[idx1], rows1, gsem)
    g0.start()
    g1.start()

    def compact(rows):
        def tok_body(i, _):
            for k in range(D // 16):
                rows[i, pl.ds(16 * k, 16)] = (
                    rows[2 * i, pl.ds(16 * k, 16)]
                    + rows[2 * i + 1, pl.ds(16 * k, 16)])
            return 0
        lax.fori_loop(0, 32, tok_body, 0)

    g0.wait()
    compact(rows0)
    w0 = pltpu.make_async_copy(rows0.at[pl.ds(0, 32)],
                               out_hbm.at[pl.ds(wid * 64, 32)], wsem)
    w0.start()
    g1.wait()
    compact(rows1)
    w1 = pltpu.make_async_copy(rows1.at[pl.ds(0, 32)],
                               out_hbm.at[pl.ds(wid * 64 + 32, 32)], wsem)
    w1.start()
    w0.wait()
    w1.wait()


_combine = functools.partial(
    pl.kernel,
    out_type=jax.ShapeDtypeStruct((T, D), jnp.float32),
    mesh=_MESH,
    scratch_types=[
        pltpu.VMEM((64,), jnp.int32),
        pltpu.VMEM((64,), jnp.int32),
        pltpu.VMEM((64, D), jnp.float32),
        pltpu.VMEM((64, D), jnp.float32),
        pltpu.SemaphoreType.DMA,
        pltpu.SemaphoreType.DMA,
    ],
)(_comb_body)


def kernel(hidden_states, topk_weights, topk_ids, gate_up_weights, down_weights):
    ids2d = topk_ids.reshape(NR, 128).astype(jnp.int32)
    w_flat = topk_weights.reshape(-1)
    tok_flat = jnp.arange(NSLOT, dtype=jnp.int32) // K

    pos2d, eot2d, rows2d = _meta(ids2d)
    pos = pos2d.reshape(NSLOT)
    eot = eot2d.reshape(128)
    rows = rows2d.reshape(128)

    x_pad, ws = _dispatch(hidden_states, pos, tok_flat, w_flat)
    g = _grouped_gu(eot, rows, x_pad, gate_up_weights, gate_up_weights)
    y = _grouped_down(eot, rows, g, down_weights, ws.reshape(NT, 1, BLOCK_M))
    out = _combine(y, pos)
    return out
